# speedup vs baseline: 2.9691x; 2.9691x over previous
import functools

import jax
import jax.numpy as jnp
from jax import lax
from jax.experimental import pallas as pl
from jax.experimental.pallas import tpu as pltpu
from jax.experimental.pallas import tpu_sc as plsc

NC, NS = 2, 16
NW = NC * NS
CHUNK = 128


def _emb_call(n, D, n_chunks, table, idx3):
    mesh = plsc.VectorSubcoreMesh(core_axis_name="c", subcore_axis_name="s")
    per_w = n_chunks * CHUNK

    @functools.partial(
        pl.kernel,
        out_type=jax.ShapeDtypeStruct((n, D), jnp.float32),
        mesh=mesh,
        scratch_types=[
            pltpu.VMEM((n_chunks, CHUNK), jnp.int32),
            pltpu.VMEM((CHUNK, D), jnp.float32),
            pltpu.SemaphoreType.DMA,
        ],
    )
    def emb(table_hbm, idx_hbm, out_hbm, idx_v, rows_v, sem):
        wid = lax.axis_index("s") * NC + lax.axis_index("c")
        base = wid * per_w
        pltpu.sync_copy(idx_hbm.at[wid], idx_v)

        def body(j, carry):
            pltpu.async_copy(table_hbm.at[idx_v.at[j]], rows_v, sem).wait()
            pltpu.sync_copy(rows_v, out_hbm.at[pl.ds(base + j * CHUNK, CHUNK)])
            return carry

        lax.fori_loop(0, n_chunks, body, 0)

    return emb(table, idx3)


def kernel(x, embed_weight):
    B, H = x.shape
    V, D = embed_weight.shape
    n = B * H
    per_w = n // NW
    n_chunks = per_w // CHUNK
    idx3 = x.reshape(NW, n_chunks, CHUNK).astype(jnp.int32)
    out = _emb_call(n, D, n_chunks, embed_weight, idx3)
    return out.reshape(B, H, D)


# speedup vs baseline: 3.3202x; 1.1182x over previous
import functools

import jax
import jax.numpy as jnp
from jax import lax
from jax.experimental import pallas as pl
from jax.experimental.pallas import tpu as pltpu
from jax.experimental.pallas import tpu_sc as plsc

NC, NS = 2, 16
NW = NC * NS
CHUNK = 128
NBUF = 5


def _emb_call(n, D, n_chunks, table, idx3):
    mesh = plsc.VectorSubcoreMesh(core_axis_name="c", subcore_axis_name="s")
    per_w = n_chunks * CHUNK
    n_super = n_chunks // NBUF

    @functools.partial(
        pl.kernel,
        out_type=jax.ShapeDtypeStruct((n, D), jnp.float32),
        mesh=mesh,
        scratch_types=[
            pltpu.VMEM((n_chunks, CHUNK), jnp.int32),
            pltpu.VMEM((NBUF, CHUNK, D), jnp.float32),
        ]
        + [pltpu.SemaphoreType.DMA] * (2 * NBUF),
    )
    def emb(table_hbm, idx_hbm, out_hbm, idx_v, rows_v, *sems):
        g_sems, w_sems = sems[:NBUF], sems[NBUF:]
        wid = lax.axis_index("s") * NC + lax.axis_index("c")
        base = wid * per_w
        pltpu.sync_copy(idx_hbm.at[wid], idx_v)

        def super_body(g, carry):
            gathers = []
            for b in range(NBUF):
                @pl.when(g > 0)
                def _():
                    pltpu.make_async_copy(
                        rows_v.at[b], out_hbm.at[pl.ds(base, CHUNK)], w_sems[b]
                    ).wait()

                j = g * NBUF + b
                gathers.append(
                    pltpu.async_copy(
                        table_hbm.at[idx_v.at[j]], rows_v.at[b], g_sems[b]
                    )
                )
            for b in range(NBUF):
                j = g * NBUF + b
                gathers[b].wait()
                pltpu.async_copy(
                    rows_v.at[b],
                    out_hbm.at[pl.ds(base + j * CHUNK, CHUNK)],
                    w_sems[b],
                )
            return carry

        lax.fori_loop(0, n_super, super_body, 0)
        for b in range(NBUF):
            pltpu.make_async_copy(
                rows_v.at[b], out_hbm.at[pl.ds(base, CHUNK)], w_sems[b]
            ).wait()

    return emb(table, idx3)


def kernel(x, embed_weight):
    B, H = x.shape
    V, D = embed_weight.shape
    n = B * H
    per_w = n // NW
    n_chunks = per_w // CHUNK
    idx3 = x.reshape(NW, n_chunks, CHUNK).astype(jnp.int32)
    out = _emb_call(n, D, n_chunks, embed_weight, idx3)
    return out.reshape(B, H, D)


# speedup vs baseline: 5.8812x; 1.7714x over previous
import functools

import jax
import jax.numpy as jnp
from jax import lax
from jax.experimental import pallas as pl
from jax.experimental.pallas import tpu as pltpu
from jax.experimental.pallas import tpu_sc as plsc

NC, NS = 2, 16
NW = NC * NS
BPC = 2
NBUF = 4


def _emb_call(B, H, D, table, idx3):
    mesh = plsc.VectorSubcoreMesh(core_axis_name="c", subcore_axis_name="s")
    b_per_w = B // NW
    n_chunks = b_per_w // BPC
    cidx = BPC * H
    n_super = n_chunks // NBUF

    @functools.partial(
        pl.kernel,
        out_type=jax.ShapeDtypeStruct((B, H, D), jnp.float32),
        mesh=mesh,
        scratch_types=[
            pltpu.VMEM((n_chunks, cidx), jnp.int32),
            pltpu.VMEM((NBUF, cidx, D), jnp.float32),
        ]
        + [pltpu.SemaphoreType.DMA] * (2 * NBUF),
    )
    def emb(table_hbm, idx_hbm, out_hbm, idx_v, rows_v, *sems):
        g_sems, w_sems = sems[:NBUF], sems[NBUF:]
        wid = lax.axis_index("s") * NC + lax.axis_index("c")
        bbase = wid * b_per_w
        pltpu.sync_copy(idx_hbm.at[wid], idx_v)

        def super_body(g, carry):
            gathers = []
            for b in range(NBUF):
                @pl.when(g > 0)
                def _():
                    for _ in range(BPC):
                        pltpu.make_async_copy(
                            rows_v.at[b, pl.ds(0, H)],
                            out_hbm.at[bbase],
                            w_sems[b],
                        ).wait()

                j = g * NBUF + b
                gathers.append(
                    pltpu.async_copy(
                        table_hbm.at[idx_v.at[j]], rows_v.at[b], g_sems[b]
                    )
                )
            for b in range(NBUF):
                j = g * NBUF + b
                gathers[b].wait()
                for r in range(BPC):
                    pltpu.async_copy(
                        rows_v.at[b, pl.ds(r * H, H)],
                        out_hbm.at[bbase + j * BPC + r],
                        w_sems[b],
                    )
            return carry

        lax.fori_loop(0, n_super, super_body, 0)
        for b in range(NBUF):
            for _ in range(BPC):
                pltpu.make_async_copy(
                    rows_v.at[b, pl.ds(0, H)], out_hbm.at[bbase], w_sems[b]
                ).wait()

    return emb(table, idx3)


def kernel(x, embed_weight):
    B, H = x.shape
    V, D = embed_weight.shape
    b_per_w = B // NW
    n_chunks = b_per_w // BPC
    idx3 = x.reshape(NW, n_chunks, BPC * H).astype(jnp.int32)
    return _emb_call(B, H, D, embed_weight, idx3)
